# Initial kernel scaffold; baseline (speedup 1.0000x reference)
import jax
import jax.numpy as jnp
from jax.experimental import pallas as pl


def kernel(x, path_2, path_3, batch, W1, b1, g1, be1, W2, b2, g2, be2, Wih, Whh, bih, bhh, gc1, bc1, gc2, bc2, Wl1, bl1, Wl2, bl2):
    raise NotImplementedError("write your pallas kernel here")


# speedup vs baseline: 1.4145x; 1.4145x over previous
import jax
import jax.numpy as jnp
from jax import lax
from jax.experimental import pallas as pl
from jax.experimental.pallas import tpu as pltpu
from jax.experimental.pallas import tpu_sc as plsc

N = 10000
H = 64
HG = 4 * H
AC = 5 * H
P = 320000
G = 64
C = 10
EPS = 1e-5

B = 128
NTILES = 32
CHUNKS = P // B
CPT = (CHUNKS + NTILES - 1) // NTILES
NSUB = 16
NA = 10240
RPT = NA // NSUB

_f32 = jnp.float32


def _dot(a, b):
    return lax.dot_general(a, b, (((1,), (1,)), ((), ())),
                           preferred_element_type=_f32)


def _bn(h, gamma, beta):
    m = jnp.mean(h, axis=0, keepdims=True)
    v = jnp.mean((h - m) ** 2, axis=0, keepdims=True)
    return gamma * (h - m) * lax.rsqrt(v + EPS) + beta


def _node_tables(h, Wih, bih, bhh, Whh, zx_ref, ac_ref):
    zx = _dot(h, Wih) + bih
    g0 = zx + bhh
    i0 = jax.nn.sigmoid(g0[:, :H])
    gg0 = jnp.tanh(g0[:, 2 * H:3 * H])
    o0 = jax.nn.sigmoid(g0[:, 3 * H:])
    c1 = i0 * gg0
    h1 = o0 * jnp.tanh(c1)
    a1 = _dot(h1, Whh) + bhh
    zx_ref[...] = zx
    ac_ref[...] = jnp.concatenate([a1, c1], axis=1)


def _encode_body(x_ref, W1_ref, b1_ref, g1_ref, be1_ref, W2_ref, b2_ref,
                 g2_ref, be2_ref, Wih_ref, bih_ref, bhh_ref, Whh_ref,
                 h_ref, zx_ref, ac_ref):
    h = jax.nn.relu(_bn(_dot(x_ref[...], W1_ref[...]) + b1_ref[...],
                        g1_ref[...], be1_ref[...]))
    h = jax.nn.relu(_bn(_dot(h, W2_ref[...]) + b2_ref[...],
                        g2_ref[...], be2_ref[...]))
    h_ref[...] = h
    _node_tables(h, Wih_ref[...], bih_ref[...], bhh_ref[...], Whh_ref[...],
                 zx_ref, ac_ref)


def _finish_tables_body(parts_ref, hprev_ref, gc_ref, bc_ref,
                        Wih_ref, bih_ref, bhh_ref, Whh_ref,
                        h_ref, zx_ref, ac_ref):
    out = parts_ref[0, :N] + parts_ref[1, :N] + hprev_ref[...]
    h = jax.nn.relu(_bn(out, gc_ref[...], bc_ref[...]))
    h_ref[...] = h
    _node_tables(h, Wih_ref[...], bih_ref[...], bhh_ref[...], Whh_ref[...],
                 zx_ref, ac_ref)


def _gates_body(h2_ref, Whh_ref, bhh_ref, out_ref):
    out_ref[...] = _dot(h2_ref[...], Whh_ref[...]) + bhh_ref[...]


def _head_body(parts_ref, hprev_ref, gc_ref, bc_ref, batch_ref,
               Wl1_ref, bl1_ref, Wl2_ref, bl2_ref, out_ref):
    out = parts_ref[0, :N] + parts_ref[1, :N] + hprev_ref[...]
    h = jax.nn.relu(_bn(out, gc_ref[...], bc_ref[...]))
    seg = lax.broadcasted_iota(jnp.int32, (G, N), 0)
    onehot = (batch_ref[...] == seg).astype(_f32)
    pooled = lax.dot_general(onehot, h, (((1,), (0,)), ((), ())),
                             preferred_element_type=_f32)
    y = jax.nn.relu(_dot(pooled, Wl1_ref[...]) + bl1_ref[...])
    out_ref[...] = _dot(y, Wl2_ref[...]) + bl2_ref[...]


def _cell16(iv, fv, gv, ov, cp):
    ei = jnp.exp(jnp.minimum(-iv, 40.0))
    ef = jnp.exp(jnp.minimum(-fv, 40.0))
    eg = jnp.exp(jnp.minimum(-2.0 * gv, 40.0))
    c = (1.0 - eg) / ((1.0 + ei) * (1.0 + eg)) + cp / (1.0 + ef)
    eo = jnp.exp(jnp.minimum(-ov, 40.0))
    ec = jnp.exp(jnp.minimum(-2.0 * c, 40.0))
    hv = (1.0 - ec) / ((1.0 + eo) * (1.0 + ec))
    return hv, c


def _sc_wid():
    return lax.axis_index("s") * 2 + lax.axis_index("c")


def _conv2_body(p0_hbm, p1_hbm, zx_hbm, ac_hbm, zero_hbm, out_hbm,
                idx0, idx1, zxg, acg, hob, acc):
    cid = lax.axis_index("c")
    sid = lax.axis_index("s")
    wid = _sc_wid()
    pltpu.sync_copy(zero_hbm, acc.at[pl.ds(sid * RPT, RPT)])
    plsc.subcore_barrier()

    def chunk(k, carry):
        cidx = wid + NTILES * k

        @pl.when(cidx < CHUNKS)
        def _():
            off = cidx * B
            pltpu.sync_copy(p0_hbm.at[pl.ds(off, B)], idx0)
            pltpu.sync_copy(p1_hbm.at[pl.ds(off, B)], idx1)
            pltpu.sync_copy(zx_hbm.at[idx1], zxg)
            pltpu.sync_copy(ac_hbm.at[idx0], acg)

            def row(r, c2):
                for j in range(4):
                    s = pl.ds(j * 16, 16)
                    iv = zxg[r, pl.ds(j * 16, 16)] + acg[r, pl.ds(j * 16, 16)]
                    fv = zxg[r, pl.ds(H + j * 16, 16)] + acg[r, pl.ds(H + j * 16, 16)]
                    gv = zxg[r, pl.ds(2 * H + j * 16, 16)] + acg[r, pl.ds(2 * H + j * 16, 16)]
                    ov = zxg[r, pl.ds(3 * H + j * 16, 16)] + acg[r, pl.ds(3 * H + j * 16, 16)]
                    cp = acg[r, pl.ds(4 * H + j * 16, 16)]
                    hv, _ = _cell16(iv, fv, gv, ov, cp)
                    hob[r, s] = hv
                return c2

            lax.fori_loop(0, B, row, 0)
            pltpu.sync_copy(hob, acc.at[idx1], add=True)

        return carry

    lax.fori_loop(0, CPT, chunk, 0)
    plsc.subcore_barrier()
    pltpu.sync_copy(acc.at[pl.ds(sid * RPT, RPT)],
                    out_hbm.at[cid, pl.ds(sid * RPT, RPT)])


def _conv3a_body(p0_hbm, p1_hbm, zx_hbm, ac_hbm, h2_hbm, c2_hbm,
                 idx0, idx1, zxg, acg, hob, cob):
    wid = _sc_wid()

    def chunk(k, carry):
        cidx = wid + NTILES * k

        @pl.when(cidx < CHUNKS)
        def _():
            off = cidx * B
            pltpu.sync_copy(p0_hbm.at[pl.ds(off, B)], idx0)
            pltpu.sync_copy(p1_hbm.at[pl.ds(off, B)], idx1)
            pltpu.sync_copy(zx_hbm.at[idx1], zxg)
            pltpu.sync_copy(ac_hbm.at[idx0], acg)

            def row(r, c2):
                for j in range(4):
                    s = pl.ds(j * 16, 16)
                    iv = zxg[r, pl.ds(j * 16, 16)] + acg[r, pl.ds(j * 16, 16)]
                    fv = zxg[r, pl.ds(H + j * 16, 16)] + acg[r, pl.ds(H + j * 16, 16)]
                    gv = zxg[r, pl.ds(2 * H + j * 16, 16)] + acg[r, pl.ds(2 * H + j * 16, 16)]
                    ov = zxg[r, pl.ds(3 * H + j * 16, 16)] + acg[r, pl.ds(3 * H + j * 16, 16)]
                    cp = acg[r, pl.ds(4 * H + j * 16, 16)]
                    hv, cv = _cell16(iv, fv, gv, ov, cp)
                    hob[r, s] = hv
                    cob[r, s] = cv
                return c2

            lax.fori_loop(0, B, row, 0)
            pltpu.sync_copy(hob, h2_hbm.at[pl.ds(off, B)])
            pltpu.sync_copy(cob, c2_hbm.at[pl.ds(off, B)])

        return carry

    lax.fori_loop(0, CPT, chunk, 0)


def _conv3b_body(p2_hbm, zx_hbm, g2_hbm, c2_hbm, zero_hbm, out_hbm,
                 idx2, zxg, g2b, c2b, hob, acc):
    cid = lax.axis_index("c")
    sid = lax.axis_index("s")
    wid = _sc_wid()
    pltpu.sync_copy(zero_hbm, acc.at[pl.ds(sid * RPT, RPT)])
    plsc.subcore_barrier()

    def chunk(k, carry):
        cidx = wid + NTILES * k

        @pl.when(cidx < CHUNKS)
        def _():
            off = cidx * B
            pltpu.sync_copy(p2_hbm.at[pl.ds(off, B)], idx2)
            pltpu.sync_copy(zx_hbm.at[idx2], zxg)
            pltpu.sync_copy(g2_hbm.at[pl.ds(off, B)], g2b)
            pltpu.sync_copy(c2_hbm.at[pl.ds(off, B)], c2b)

            def row(r, c2):
                for j in range(4):
                    s = pl.ds(j * 16, 16)
                    iv = zxg[r, pl.ds(j * 16, 16)] + g2b[r, pl.ds(j * 16, 16)]
                    fv = zxg[r, pl.ds(H + j * 16, 16)] + g2b[r, pl.ds(H + j * 16, 16)]
                    gv = zxg[r, pl.ds(2 * H + j * 16, 16)] + g2b[r, pl.ds(2 * H + j * 16, 16)]
                    ov = zxg[r, pl.ds(3 * H + j * 16, 16)] + g2b[r, pl.ds(3 * H + j * 16, 16)]
                    cp = c2b[r, s]
                    hv, _ = _cell16(iv, fv, gv, ov, cp)
                    hob[r, s] = hv
                return c2

            lax.fori_loop(0, B, row, 0)
            pltpu.sync_copy(hob, acc.at[idx2], add=True)

        return carry

    lax.fori_loop(0, CPT, chunk, 0)
    plsc.subcore_barrier()
    pltpu.sync_copy(acc.at[pl.ds(sid * RPT, RPT)],
                    out_hbm.at[cid, pl.ds(sid * RPT, RPT)])


def _sc_mesh():
    return plsc.VectorSubcoreMesh(core_axis_name="c", subcore_axis_name="s")


def kernel(x, path_2, path_3, batch, W1, b1, g1, be1, W2, b2, g2, be2,
           Wih, Whh, bih, bhh, gc1, bc1, gc2, bc2, Wl1, bl1, Wl2, bl2):
    r2 = lambda v: v.reshape(1, -1)

    sd = jax.ShapeDtypeStruct
    h0, zx1, ac1 = pl.pallas_call(
        _encode_body,
        out_shape=[sd((N, H), _f32), sd((N, HG), _f32), sd((N, AC), _f32)],
    )(x, W1, r2(b1), r2(g1), r2(be1), W2, r2(b2), r2(g2), r2(be2),
      Wih, r2(bih), r2(bhh), Whh)

    zero = jnp.zeros((RPT, H), _f32)

    conv2 = pl.kernel(
        _conv2_body,
        out_type=sd((2, NA, H), _f32),
        mesh=_sc_mesh(),
        compiler_params=pltpu.CompilerParams(use_tc_tiling_on_sc=False),
        scratch_types=[
            pltpu.VMEM((B,), jnp.int32),
            pltpu.VMEM((B,), jnp.int32),
            pltpu.VMEM((B, HG), _f32),
            pltpu.VMEM((B, AC), _f32),
            pltpu.VMEM((B, H), _f32),
            pltpu.VMEM_SHARED((N, H), _f32),
        ],
    )
    parts1 = conv2(path_2[:, 0], path_2[:, 1], zx1, ac1, zero)

    h1, zx2, ac2 = pl.pallas_call(
        _finish_tables_body,
        out_shape=[sd((N, H), _f32), sd((N, HG), _f32), sd((N, AC), _f32)],
    )(parts1, h0, r2(gc1), r2(bc1), Wih, r2(bih), r2(bhh), Whh)

    conv3a = pl.kernel(
        _conv3a_body,
        out_type=[sd((P, H), _f32), sd((P, H), _f32)],
        mesh=_sc_mesh(),
        compiler_params=pltpu.CompilerParams(use_tc_tiling_on_sc=False),
        scratch_types=[
            pltpu.VMEM((B,), jnp.int32),
            pltpu.VMEM((B,), jnp.int32),
            pltpu.VMEM((B, HG), _f32),
            pltpu.VMEM((B, AC), _f32),
            pltpu.VMEM((B, H), _f32),
            pltpu.VMEM((B, H), _f32),
        ],
    )
    h2p, c2p = conv3a(path_3[:, 0], path_3[:, 1], zx2, ac2)

    RB = 4000
    g2m = pl.pallas_call(
        _gates_body,
        grid=(P // RB,),
        in_specs=[
            pl.BlockSpec((RB, H), lambda i: (i, 0)),
            pl.BlockSpec((HG, H), lambda i: (0, 0)),
            pl.BlockSpec((1, HG), lambda i: (0, 0)),
        ],
        out_specs=pl.BlockSpec((RB, HG), lambda i: (i, 0)),
        out_shape=sd((P, HG), _f32),
    )(h2p, Whh, r2(bhh))

    conv3b = pl.kernel(
        _conv3b_body,
        out_type=sd((2, NA, H), _f32),
        mesh=_sc_mesh(),
        compiler_params=pltpu.CompilerParams(use_tc_tiling_on_sc=False),
        scratch_types=[
            pltpu.VMEM((B,), jnp.int32),
            pltpu.VMEM((B, HG), _f32),
            pltpu.VMEM((B, HG), _f32),
            pltpu.VMEM((B, H), _f32),
            pltpu.VMEM((B, H), _f32),
            pltpu.VMEM_SHARED((N, H), _f32),
        ],
    )
    parts2 = conv3b(path_3[:, 2], zx2, g2m, c2p, zero)

    out = pl.pallas_call(
        _head_body,
        out_shape=sd((G, C), _f32),
    )(parts2, h1, r2(gc2), r2(bc2), r2(batch), Wl1, r2(bl1), Wl2, r2(bl2))
    return out


# speedup vs baseline: 1.7113x; 1.2098x over previous
import jax
import jax.numpy as jnp
from jax import lax
from jax.experimental import pallas as pl
from jax.experimental.pallas import tpu as pltpu
from jax.experimental.pallas import tpu_sc as plsc

N = 10000
H = 64
HG = 4 * H
AC = 5 * H
P = 320000
G = 64
C = 10
EPS = 1e-5

NT = 32
B = 48
CT = 212
PP = NT * CT * B
PAD = PP - P
PT = CT * B
NSUB = 16
NA = 10112
RPT = NA // NSUB

_f32 = jnp.float32


def _dot(a, b):
    return lax.dot_general(a, b, (((1,), (1,)), ((), ())),
                           preferred_element_type=_f32)


def _bn(h, gamma, beta):
    m = jnp.mean(h, axis=0, keepdims=True)
    v = jnp.mean((h - m) ** 2, axis=0, keepdims=True)
    return gamma * (h - m) * lax.rsqrt(v + EPS) + beta


def _node_tables(h, Wih, bih, bhh, Whh, zx_ref, ac_ref):
    zx = _dot(h, Wih) + bih
    g0 = zx + bhh
    i0 = jax.nn.sigmoid(g0[:, :H])
    gg0 = jnp.tanh(g0[:, 2 * H:3 * H])
    o0 = jax.nn.sigmoid(g0[:, 3 * H:])
    c1 = i0 * gg0
    h1 = o0 * jnp.tanh(c1)
    a1 = _dot(h1, Whh) + bhh
    zx_ref[...] = jnp.concatenate([zx, jnp.zeros((NA - N, HG), _f32)], axis=0)
    ac_ref[...] = jnp.concatenate(
        [jnp.concatenate([a1, c1], axis=1), jnp.zeros((NA - N, AC), _f32)],
        axis=0)


def _encode_body(x_ref, W1_ref, b1_ref, g1_ref, be1_ref, W2_ref, b2_ref,
                 g2_ref, be2_ref, Wih_ref, bih_ref, bhh_ref, Whh_ref,
                 h_ref, zx_ref, ac_ref):
    h = jax.nn.relu(_bn(_dot(x_ref[...], W1_ref[...]) + b1_ref[...],
                        g1_ref[...], be1_ref[...]))
    h = jax.nn.relu(_bn(_dot(h, W2_ref[...]) + b2_ref[...],
                        g2_ref[...], be2_ref[...]))
    h_ref[...] = h
    _node_tables(h, Wih_ref[...], bih_ref[...], bhh_ref[...], Whh_ref[...],
                 zx_ref, ac_ref)


def _finish_tables_body(parts_ref, hprev_ref, gc_ref, bc_ref,
                        Wih_ref, bih_ref, bhh_ref, Whh_ref,
                        h_ref, zx_ref, ac_ref):
    out = parts_ref[0, :N] + parts_ref[1, :N] + hprev_ref[...]
    h = jax.nn.relu(_bn(out, gc_ref[...], bc_ref[...]))
    h_ref[...] = h
    _node_tables(h, Wih_ref[...], bih_ref[...], bhh_ref[...], Whh_ref[...],
                 zx_ref, ac_ref)


def _gates_body(h2_ref, Whh_ref, bhh_ref, out_ref):
    out_ref[...] = _dot(h2_ref[...], Whh_ref[...]) + bhh_ref[...]


def _head_body(parts_ref, hprev_ref, gc_ref, bc_ref, batch_ref,
               Wl1_ref, bl1_ref, Wl2_ref, bl2_ref, out_ref):
    out = parts_ref[0, :N] + parts_ref[1, :N] + hprev_ref[...]
    h = jax.nn.relu(_bn(out, gc_ref[...], bc_ref[...]))
    seg = lax.broadcasted_iota(jnp.int32, (G, N), 0)
    onehot = (batch_ref[...] == seg).astype(_f32)
    pooled = lax.dot_general(onehot, h, (((1,), (0,)), ((), ())),
                             preferred_element_type=_f32)
    y = jax.nn.relu(_dot(pooled, Wl1_ref[...]) + bl1_ref[...])
    out_ref[...] = _dot(y, Wl2_ref[...]) + bl2_ref[...]


def _cell16(iv, fv, gv, ov, cp):
    ei = jnp.exp(jnp.minimum(-iv, 40.0))
    ef = jnp.exp(jnp.minimum(-fv, 40.0))
    eg = jnp.exp(jnp.minimum(-2.0 * gv, 40.0))
    c = (1.0 - eg) / ((1.0 + ei) * (1.0 + eg)) + cp / (1.0 + ef)
    eo = jnp.exp(jnp.minimum(-ov, 40.0))
    ec = jnp.exp(-2.0 * c)
    hv = (1.0 - ec) / ((1.0 + eo) * (1.0 + ec))
    return hv, c


def _rows(zxg, og, hob, cob, cp_at):
    def row(r, carry):
        for j in range(4):
            sl = pl.ds(j * 16, 16)
            iv = zxg[r, pl.ds(j * 16, 16)] + og[r, pl.ds(j * 16, 16)]
            fv = zxg[r, pl.ds(H + j * 16, 16)] + og[r, pl.ds(H + j * 16, 16)]
            gv = zxg[r, pl.ds(2 * H + j * 16, 16)] + og[r, pl.ds(2 * H + j * 16, 16)]
            ov = zxg[r, pl.ds(3 * H + j * 16, 16)] + og[r, pl.ds(3 * H + j * 16, 16)]
            hv, cv = _cell16(iv, fv, gv, ov, cp_at(r, j))
            hob[r, sl] = hv
            if cob is not None:
                cob[r, sl] = cv
        return carry
    lax.fori_loop(0, B, row, 0, unroll=2)


def _sc_wid():
    return lax.axis_index("s") * 2 + lax.axis_index("c")


def _conv2_body(p0g_hbm, p1g_hbm, zx_hbm, ac_hbm, zero_hbm, out_hbm,
                i0, i1, zxg0, zxg1, acg0, acg1, hob0, hob1, acc,
                sz0, sz1, sa0, sa1, ss0, ss1):
    cid = lax.axis_index("c")
    sid = lax.axis_index("s")
    wid = _sc_wid()
    pltpu.sync_copy(zero_hbm, acc.at[pl.ds(sid * RPT, RPT)])
    pltpu.sync_copy(p0g_hbm.at[wid], i0)
    pltpu.sync_copy(p1g_hbm.at[wid], i1)
    plsc.subcore_barrier()

    bufs = ((zxg0, acg0, hob0, sz0, sa0, ss0), (zxg1, acg1, hob1, sz1, sa1, ss1))

    def gathers(c, bi):
        zxg, acg, _, sz, sa, _ = bufs[bi]
        pltpu.async_copy(zx_hbm.at[i1.at[c]], zxg, sz)
        pltpu.async_copy(ac_hbm.at[i0.at[c]], acg, sa)

    gathers(0, 0)
    gathers(1, 1)

    def step(i, carry):
        for bi in range(2):
            zxg, acg, hob, sz, sa, ss = bufs[bi]
            c = 2 * i + bi
            pltpu.make_async_copy(zx_hbm.at[i1.at[c]], zxg, sz).wait()
            pltpu.make_async_copy(ac_hbm.at[i0.at[c]], acg, sa).wait()

            @pl.when(c >= 2)
            def _():
                pltpu.make_async_copy(hob, acc.at[i1.at[c]], ss).wait()

            _rows(zxg, acg, hob, None,
                  lambda r, j: acg[r, pl.ds(4 * H + j * 16, 16)])
            pltpu.async_copy(hob, acc.at[i1.at[c]], ss, add=True)

            @pl.when(c + 2 < CT)
            def _():
                gathers(c + 2, bi)
        return carry

    lax.fori_loop(0, CT // 2, step, 0)
    for bi in range(2):
        _, _, hob, _, _, ss = bufs[bi]
        pltpu.make_async_copy(hob, acc.at[i1.at[CT - 2 + bi]], ss).wait()
    plsc.subcore_barrier()
    pltpu.sync_copy(acc.at[pl.ds(sid * RPT, RPT)],
                    out_hbm.at[cid, pl.ds(sid * RPT, RPT)])


def _conv3a_body(p0g_hbm, p1g_hbm, zx_hbm, ac_hbm, h2_hbm, c2_hbm,
                 i0, i1, zxg0, zxg1, acg0, acg1, hob0, hob1, cob0, cob1,
                 sz0, sz1, sa0, sa1, sw0, sw1):
    wid = _sc_wid()
    pltpu.sync_copy(p0g_hbm.at[wid], i0)
    pltpu.sync_copy(p1g_hbm.at[wid], i1)

    bufs = ((zxg0, acg0, hob0, cob0, sz0, sa0, sw0),
            (zxg1, acg1, hob1, cob1, sz1, sa1, sw1))

    def gathers(c, bi):
        zxg, acg, _, _, sz, sa, _ = bufs[bi]
        pltpu.async_copy(zx_hbm.at[i1.at[c]], zxg, sz)
        pltpu.async_copy(ac_hbm.at[i0.at[c]], acg, sa)

    gathers(0, 0)
    gathers(1, 1)

    def step(i, carry):
        for bi in range(2):
            zxg, acg, hob, cob, sz, sa, sw = bufs[bi]
            c = 2 * i + bi
            off = wid * PT + c * B
            pltpu.make_async_copy(zx_hbm.at[i1.at[c]], zxg, sz).wait()
            pltpu.make_async_copy(ac_hbm.at[i0.at[c]], acg, sa).wait()

            @pl.when(c >= 2)
            def _():
                pltpu.make_async_copy(hob, h2_hbm.at[pl.ds(off - 2 * B, B)],
                                      sw).wait()
                pltpu.make_async_copy(cob, c2_hbm.at[pl.ds(off - 2 * B, B)],
                                      sw).wait()

            _rows(zxg, acg, hob, cob,
                  lambda r, j: acg[r, pl.ds(4 * H + j * 16, 16)])
            pltpu.async_copy(hob, h2_hbm.at[pl.ds(off, B)], sw)
            pltpu.async_copy(cob, c2_hbm.at[pl.ds(off, B)], sw)

            @pl.when(c + 2 < CT)
            def _():
                gathers(c + 2, bi)
        return carry

    lax.fori_loop(0, CT // 2, step, 0)
    for bi in range(2):
        _, _, hob, cob, _, _, sw = bufs[bi]
        off = wid * PT + (CT - 2 + bi) * B
        pltpu.make_async_copy(hob, h2_hbm.at[pl.ds(off, B)], sw).wait()
        pltpu.make_async_copy(cob, c2_hbm.at[pl.ds(off, B)], sw).wait()


def _conv3b_body(p2g_hbm, zx_hbm, g2_hbm, c2_hbm, zero_hbm, out_hbm,
                 i2, zxg0, zxg1, g2b0, g2b1, c2b0, c2b1, hob0, hob1, acc,
                 sz0, sz1, sg0, sg1, sc0, sc1, ss0, ss1):
    cid = lax.axis_index("c")
    sid = lax.axis_index("s")
    wid = _sc_wid()
    pltpu.sync_copy(zero_hbm, acc.at[pl.ds(sid * RPT, RPT)])
    pltpu.sync_copy(p2g_hbm.at[wid], i2)
    plsc.subcore_barrier()

    bufs = ((zxg0, g2b0, c2b0, hob0, sz0, sg0, sc0, ss0),
            (zxg1, g2b1, c2b1, hob1, sz1, sg1, sc1, ss1))

    def loads(c, bi):
        zxg, g2b, c2b, _, sz, sg, sc, _ = bufs[bi]
        off = wid * PT + c * B
        pltpu.async_copy(zx_hbm.at[i2.at[c]], zxg, sz)
        pltpu.async_copy(g2_hbm.at[pl.ds(off, B)], g2b, sg)
        pltpu.async_copy(c2_hbm.at[pl.ds(off, B)], c2b, sc)

    loads(0, 0)
    loads(1, 1)

    def step(i, carry):
        for bi in range(2):
            zxg, g2b, c2b, hob, sz, sg, sc, ss = bufs[bi]
            c = 2 * i + bi
            off = wid * PT + c * B
            pltpu.make_async_copy(zx_hbm.at[i2.at[c]], zxg, sz).wait()
            pltpu.make_async_copy(g2_hbm.at[pl.ds(off, B)], g2b, sg).wait()
            pltpu.make_async_copy(c2_hbm.at[pl.ds(off, B)], c2b, sc).wait()

            @pl.when(c >= 2)
            def _():
                pltpu.make_async_copy(hob, acc.at[i2.at[c]], ss).wait()

            _rows(zxg, g2b, hob, None,
                  lambda r, j: c2b[r, pl.ds(j * 16, 16)])
            pltpu.async_copy(hob, acc.at[i2.at[c]], ss, add=True)

            @pl.when(c + 2 < CT)
            def _():
                loads(c + 2, bi)
        return carry

    lax.fori_loop(0, CT // 2, step, 0)
    for bi in range(2):
        _, _, _, hob, _, _, _, ss = bufs[bi]
        pltpu.make_async_copy(hob, acc.at[i2.at[CT - 2 + bi]], ss).wait()
    plsc.subcore_barrier()
    pltpu.sync_copy(acc.at[pl.ds(sid * RPT, RPT)],
                    out_hbm.at[cid, pl.ds(sid * RPT, RPT)])


def _sc_mesh():
    return plsc.VectorSubcoreMesh(core_axis_name="c", subcore_axis_name="s")


_SC_PARAMS = dict(
    compiler_params=pltpu.CompilerParams(use_tc_tiling_on_sc=False))


def _pad_idx(col, padval):
    return jnp.concatenate(
        [col, jnp.full((PAD,), padval, jnp.int32)]).reshape(NT, CT, B)


def kernel(x, path_2, path_3, batch, W1, b1, g1, be1, W2, b2, g2, be2,
           Wih, Whh, bih, bhh, gc1, bc1, gc2, bc2, Wl1, bl1, Wl2, bl2):
    r2 = lambda v: v.reshape(1, -1)

    sd = jax.ShapeDtypeStruct
    h0, zx1, ac1 = pl.pallas_call(
        _encode_body,
        out_shape=[sd((N, H), _f32), sd((NA, HG), _f32), sd((NA, AC), _f32)],
    )(x, W1, r2(b1), r2(g1), r2(be1), W2, r2(b2), r2(g2), r2(be2),
      Wih, r2(bih), r2(bhh), Whh)

    zero = jnp.zeros((RPT, H), _f32)
    dma = pltpu.SemaphoreType.DMA

    conv2 = pl.kernel(
        _conv2_body,
        out_type=sd((2, NA, H), _f32),
        mesh=_sc_mesh(),
        scratch_types=[
            pltpu.VMEM((CT, B), jnp.int32),
            pltpu.VMEM((CT, B), jnp.int32),
            pltpu.VMEM((B, HG), _f32), pltpu.VMEM((B, HG), _f32),
            pltpu.VMEM((B, AC), _f32), pltpu.VMEM((B, AC), _f32),
            pltpu.VMEM((B, H), _f32), pltpu.VMEM((B, H), _f32),
            pltpu.VMEM_SHARED((NA, H), _f32),
            dma, dma, dma, dma, dma, dma,
        ],
        **_SC_PARAMS,
    )
    parts1 = conv2(_pad_idx(path_2[:, 0], NA - 1), _pad_idx(path_2[:, 1], NA - 1),
                   zx1, ac1, zero)

    h1, zx2, ac2 = pl.pallas_call(
        _finish_tables_body,
        out_shape=[sd((N, H), _f32), sd((NA, HG), _f32), sd((NA, AC), _f32)],
    )(parts1, h0, r2(gc1), r2(bc1), Wih, r2(bih), r2(bhh), Whh)

    conv3a = pl.kernel(
        _conv3a_body,
        out_type=[sd((PP, H), _f32), sd((PP, H), _f32)],
        mesh=_sc_mesh(),
        scratch_types=[
            pltpu.VMEM((CT, B), jnp.int32),
            pltpu.VMEM((CT, B), jnp.int32),
            pltpu.VMEM((B, HG), _f32), pltpu.VMEM((B, HG), _f32),
            pltpu.VMEM((B, AC), _f32), pltpu.VMEM((B, AC), _f32),
            pltpu.VMEM((B, H), _f32), pltpu.VMEM((B, H), _f32),
            pltpu.VMEM((B, H), _f32), pltpu.VMEM((B, H), _f32),
            dma, dma, dma, dma, dma, dma,
        ],
        **_SC_PARAMS,
    )
    h2p, c2p = conv3a(_pad_idx(path_3[:, 0], NA - 1),
                      _pad_idx(path_3[:, 1], NA - 1), zx2, ac2)

    RB = 2048
    g2m = pl.pallas_call(
        _gates_body,
        grid=(PP // RB,),
        in_specs=[
            pl.BlockSpec((RB, H), lambda i: (i, 0)),
            pl.BlockSpec((HG, H), lambda i: (0, 0)),
            pl.BlockSpec((1, HG), lambda i: (0, 0)),
        ],
        out_specs=pl.BlockSpec((RB, HG), lambda i: (i, 0)),
        out_shape=sd((PP, HG), _f32),
    )(h2p, Whh, r2(bhh))

    conv3b = pl.kernel(
        _conv3b_body,
        out_type=sd((2, NA, H), _f32),
        mesh=_sc_mesh(),
        scratch_types=[
            pltpu.VMEM((CT, B), jnp.int32),
            pltpu.VMEM((B, HG), _f32), pltpu.VMEM((B, HG), _f32),
            pltpu.VMEM((B, HG), _f32), pltpu.VMEM((B, HG), _f32),
            pltpu.VMEM((B, H), _f32), pltpu.VMEM((B, H), _f32),
            pltpu.VMEM((B, H), _f32), pltpu.VMEM((B, H), _f32),
            pltpu.VMEM_SHARED((NA, H), _f32),
            dma, dma, dma, dma, dma, dma, dma, dma,
        ],
        **_SC_PARAMS,
    )
    parts2 = conv3b(_pad_idx(path_3[:, 2], NA - 1), zx2, g2m, c2p, zero)

    out = pl.pallas_call(
        _head_body,
        out_shape=sd((G, C), _f32),
    )(parts2, h1, r2(gc2), r2(bc2), r2(batch), Wl1, r2(bl1), Wl2, r2(bl2))
    return out


# speedup vs baseline: 3.7828x; 2.2105x over previous
import jax
import jax.numpy as jnp
from jax import lax
from jax.experimental import pallas as pl
from jax.experimental.pallas import tpu as pltpu
from jax.experimental.pallas import tpu_sc as plsc

N = 10000
H = 64
HG = 4 * H
AC = 5 * H
P = 320000
G = 64
C = 10
EPS = 1e-5

NT = 32
B = 48
CT = 212
PP = NT * CT * B
PAD = PP - P
PT = CT * B
NSUB = 16
NA = 10112
RPT = NA // NSUB

_f32 = jnp.float32


def _dot(a, b):
    return lax.dot_general(a, b, (((1,), (1,)), ((), ())),
                           preferred_element_type=_f32)


def _bn(h, gamma, beta):
    m = jnp.mean(h, axis=0, keepdims=True)
    v = jnp.mean((h - m) ** 2, axis=0, keepdims=True)
    return gamma * (h - m) * lax.rsqrt(v + EPS) + beta


def _node_tables(h, Wih, bih, bhh, Whh, zx_ref, ac_ref):
    zx = _dot(h, Wih) + bih
    g0 = zx + bhh
    i0 = jax.nn.sigmoid(g0[:, :H])
    gg0 = jnp.tanh(g0[:, 2 * H:3 * H])
    o0 = jax.nn.sigmoid(g0[:, 3 * H:])
    c1 = i0 * gg0
    h1 = o0 * jnp.tanh(c1)
    a1 = _dot(h1, Whh) + bhh
    zx_ref[...] = jnp.concatenate([zx, jnp.zeros((NA - N, HG), _f32)], axis=0)
    ac_ref[...] = jnp.concatenate(
        [jnp.concatenate([a1, c1], axis=1), jnp.zeros((NA - N, AC), _f32)],
        axis=0)


def _encode_body(x_ref, W1_ref, b1_ref, g1_ref, be1_ref, W2_ref, b2_ref,
                 g2_ref, be2_ref, Wih_ref, bih_ref, bhh_ref, Whh_ref,
                 h_ref, zx_ref, ac_ref):
    h = jax.nn.relu(_bn(_dot(x_ref[...], W1_ref[...]) + b1_ref[...],
                        g1_ref[...], be1_ref[...]))
    h = jax.nn.relu(_bn(_dot(h, W2_ref[...]) + b2_ref[...],
                        g2_ref[...], be2_ref[...]))
    h_ref[...] = h
    _node_tables(h, Wih_ref[...], bih_ref[...], bhh_ref[...], Whh_ref[...],
                 zx_ref, ac_ref)


def _finish_tables_body(parts_ref, hprev_ref, gc_ref, bc_ref,
                        Wih_ref, bih_ref, bhh_ref, Whh_ref,
                        h_ref, zx_ref, ac_ref):
    out = parts_ref[0, :N] + parts_ref[1, :N] + hprev_ref[...]
    h = jax.nn.relu(_bn(out, gc_ref[...], bc_ref[...]))
    h_ref[...] = h
    _node_tables(h, Wih_ref[...], bih_ref[...], bhh_ref[...], Whh_ref[...],
                 zx_ref, ac_ref)


def _gates_body(h2_ref, Whh_ref, bhh_ref, out_ref):
    out_ref[...] = _dot(h2_ref[...], Whh_ref[...]) + bhh_ref[...]


def _head_body(parts_ref, hprev_ref, gc_ref, bc_ref, batch_ref,
               Wl1_ref, bl1_ref, Wl2_ref, bl2_ref, out_ref):
    out = parts_ref[0, :N] + parts_ref[1, :N] + hprev_ref[...]
    h = jax.nn.relu(_bn(out, gc_ref[...], bc_ref[...]))
    seg = lax.broadcasted_iota(jnp.int32, (G, N), 0)
    onehot = (batch_ref[...] == seg).astype(_f32)
    pooled = lax.dot_general(onehot, h, (((1,), (0,)), ((), ())),
                             preferred_element_type=_f32)
    y = jax.nn.relu(_dot(pooled, Wl1_ref[...]) + bl1_ref[...])
    out_ref[...] = _dot(y, Wl2_ref[...]) + bl2_ref[...]


def _cell16(iv, fv, gv, ov, cp):
    ei = jnp.exp(jnp.minimum(-iv, 40.0))
    ef = jnp.exp(jnp.minimum(-fv, 40.0))
    eg = jnp.exp(jnp.minimum(-2.0 * gv, 40.0))
    c = (1.0 - eg) / ((1.0 + ei) * (1.0 + eg)) + cp / (1.0 + ef)
    eo = jnp.exp(jnp.minimum(-ov, 40.0))
    ec = jnp.exp(-2.0 * c)
    hv = (1.0 - ec) / ((1.0 + eo) * (1.0 + ec))
    return hv, c


def _rows(zxg, og, hob, cob, cp_at):
    @plsc.parallel_loop(0, B, step=1, unroll=4)
    def row(r):
        for j in range(4):
            sl = pl.ds(j * 16, 16)
            iv = zxg[r, pl.ds(j * 16, 16)] + og[r, pl.ds(j * 16, 16)]
            fv = zxg[r, pl.ds(H + j * 16, 16)] + og[r, pl.ds(H + j * 16, 16)]
            gv = zxg[r, pl.ds(2 * H + j * 16, 16)] + og[r, pl.ds(2 * H + j * 16, 16)]
            ov = zxg[r, pl.ds(3 * H + j * 16, 16)] + og[r, pl.ds(3 * H + j * 16, 16)]
            hv, cv = _cell16(iv, fv, gv, ov, cp_at(r, j))
            hob[r, sl] = hv
            if cob is not None:
                cob[r, sl] = cv


def _sc_wid():
    return lax.axis_index("s") * 2 + lax.axis_index("c")


def _conv2_body(p0g_hbm, p1g_hbm, zx_hbm, ac_hbm, zero_hbm, out_hbm,
                i0, i1, zxg0, zxg1, acg0, acg1, hob0, hob1, acc,
                sz0, sz1, sa0, sa1, ss0, ss1):
    cid = lax.axis_index("c")
    sid = lax.axis_index("s")
    wid = _sc_wid()
    pltpu.sync_copy(zero_hbm, acc.at[pl.ds(sid * RPT, RPT)])
    pltpu.sync_copy(p0g_hbm.at[wid], i0)
    pltpu.sync_copy(p1g_hbm.at[wid], i1)
    plsc.subcore_barrier()

    bufs = ((zxg0, acg0, hob0, sz0, sa0, ss0), (zxg1, acg1, hob1, sz1, sa1, ss1))

    def gathers(c, bi):
        zxg, acg, _, sz, sa, _ = bufs[bi]
        pltpu.async_copy(zx_hbm.at[i1.at[c]], zxg, sz)
        pltpu.async_copy(ac_hbm.at[i0.at[c]], acg, sa)

    gathers(0, 0)
    gathers(1, 1)

    def step(i, carry):
        for bi in range(2):
            zxg, acg, hob, sz, sa, ss = bufs[bi]
            c = 2 * i + bi
            pltpu.make_async_copy(zx_hbm.at[i1.at[c]], zxg, sz).wait()
            pltpu.make_async_copy(ac_hbm.at[i0.at[c]], acg, sa).wait()

            @pl.when(c >= 2)
            def _():
                pltpu.make_async_copy(hob, acc.at[i1.at[c]], ss).wait()

            _rows(zxg, acg, hob, None,
                  lambda r, j: acg[r, pl.ds(4 * H + j * 16, 16)])
            pltpu.async_copy(hob, acc.at[i1.at[c]], ss, add=True)

            @pl.when(c + 2 < CT)
            def _():
                gathers(c + 2, bi)
        return carry

    lax.fori_loop(0, CT // 2, step, 0)
    for bi in range(2):
        _, _, hob, _, _, ss = bufs[bi]
        pltpu.make_async_copy(hob, acc.at[i1.at[CT - 2 + bi]], ss).wait()
    plsc.subcore_barrier()
    pltpu.sync_copy(acc.at[pl.ds(sid * RPT, RPT)],
                    out_hbm.at[cid, pl.ds(sid * RPT, RPT)])


def _conv3a_body(p0g_hbm, p1g_hbm, zx_hbm, ac_hbm, h2_hbm, c2_hbm,
                 i0, i1, zxg0, zxg1, acg0, acg1, hob0, hob1, cob0, cob1,
                 sz0, sz1, sa0, sa1, sw0, sw1):
    wid = _sc_wid()
    pltpu.sync_copy(p0g_hbm.at[wid], i0)
    pltpu.sync_copy(p1g_hbm.at[wid], i1)

    bufs = ((zxg0, acg0, hob0, cob0, sz0, sa0, sw0),
            (zxg1, acg1, hob1, cob1, sz1, sa1, sw1))

    def gathers(c, bi):
        zxg, acg, _, _, sz, sa, _ = bufs[bi]
        pltpu.async_copy(zx_hbm.at[i1.at[c]], zxg, sz)
        pltpu.async_copy(ac_hbm.at[i0.at[c]], acg, sa)

    gathers(0, 0)
    gathers(1, 1)

    def step(i, carry):
        for bi in range(2):
            zxg, acg, hob, cob, sz, sa, sw = bufs[bi]
            c = 2 * i + bi
            off = wid * PT + c * B
            pltpu.make_async_copy(zx_hbm.at[i1.at[c]], zxg, sz).wait()
            pltpu.make_async_copy(ac_hbm.at[i0.at[c]], acg, sa).wait()

            @pl.when(c >= 2)
            def _():
                pltpu.make_async_copy(hob, h2_hbm.at[pl.ds(off - 2 * B, B)],
                                      sw).wait()
                pltpu.make_async_copy(cob, c2_hbm.at[pl.ds(off - 2 * B, B)],
                                      sw).wait()

            _rows(zxg, acg, hob, cob,
                  lambda r, j: acg[r, pl.ds(4 * H + j * 16, 16)])
            pltpu.async_copy(hob, h2_hbm.at[pl.ds(off, B)], sw)
            pltpu.async_copy(cob, c2_hbm.at[pl.ds(off, B)], sw)

            @pl.when(c + 2 < CT)
            def _():
                gathers(c + 2, bi)
        return carry

    lax.fori_loop(0, CT // 2, step, 0)
    for bi in range(2):
        _, _, hob, cob, _, _, sw = bufs[bi]
        off = wid * PT + (CT - 2 + bi) * B
        pltpu.make_async_copy(hob, h2_hbm.at[pl.ds(off, B)], sw).wait()
        pltpu.make_async_copy(cob, c2_hbm.at[pl.ds(off, B)], sw).wait()


def _conv3b_body(p2g_hbm, zx_hbm, g2_hbm, c2_hbm, zero_hbm, out_hbm,
                 i2, zxg0, zxg1, g2b0, g2b1, c2b0, c2b1, hob0, hob1, acc,
                 sz0, sz1, sg0, sg1, sc0, sc1, ss0, ss1):
    cid = lax.axis_index("c")
    sid = lax.axis_index("s")
    wid = _sc_wid()
    pltpu.sync_copy(zero_hbm, acc.at[pl.ds(sid * RPT, RPT)])
    pltpu.sync_copy(p2g_hbm.at[wid], i2)
    plsc.subcore_barrier()

    bufs = ((zxg0, g2b0, c2b0, hob0, sz0, sg0, sc0, ss0),
            (zxg1, g2b1, c2b1, hob1, sz1, sg1, sc1, ss1))

    def loads(c, bi):
        zxg, g2b, c2b, _, sz, sg, sc, _ = bufs[bi]
        off = wid * PT + c * B
        pltpu.async_copy(zx_hbm.at[i2.at[c]], zxg, sz)
        pltpu.async_copy(g2_hbm.at[pl.ds(off, B)], g2b, sg)
        pltpu.async_copy(c2_hbm.at[pl.ds(off, B)], c2b, sc)

    loads(0, 0)
    loads(1, 1)

    def step(i, carry):
        for bi in range(2):
            zxg, g2b, c2b, hob, sz, sg, sc, ss = bufs[bi]
            c = 2 * i + bi
            off = wid * PT + c * B
            pltpu.make_async_copy(zx_hbm.at[i2.at[c]], zxg, sz).wait()
            pltpu.make_async_copy(g2_hbm.at[pl.ds(off, B)], g2b, sg).wait()
            pltpu.make_async_copy(c2_hbm.at[pl.ds(off, B)], c2b, sc).wait()

            @pl.when(c >= 2)
            def _():
                pltpu.make_async_copy(hob, acc.at[i2.at[c]], ss).wait()

            _rows(zxg, g2b, hob, None,
                  lambda r, j: c2b[r, pl.ds(j * 16, 16)])
            pltpu.async_copy(hob, acc.at[i2.at[c]], ss, add=True)

            @pl.when(c + 2 < CT)
            def _():
                loads(c + 2, bi)
        return carry

    lax.fori_loop(0, CT // 2, step, 0)
    for bi in range(2):
        _, _, _, hob, _, _, _, ss = bufs[bi]
        pltpu.make_async_copy(hob, acc.at[i2.at[CT - 2 + bi]], ss).wait()
    plsc.subcore_barrier()
    pltpu.sync_copy(acc.at[pl.ds(sid * RPT, RPT)],
                    out_hbm.at[cid, pl.ds(sid * RPT, RPT)])


def _sc_mesh():
    return plsc.VectorSubcoreMesh(core_axis_name="c", subcore_axis_name="s")


_SC_PARAMS = dict(
    compiler_params=pltpu.CompilerParams(use_tc_tiling_on_sc=False))


def _pad_idx(col, padval):
    return jnp.concatenate(
        [col, jnp.full((PAD,), padval, jnp.int32)]).reshape(NT, CT, B)


def kernel(x, path_2, path_3, batch, W1, b1, g1, be1, W2, b2, g2, be2,
           Wih, Whh, bih, bhh, gc1, bc1, gc2, bc2, Wl1, bl1, Wl2, bl2):
    r2 = lambda v: v.reshape(1, -1)

    sd = jax.ShapeDtypeStruct
    h0, zx1, ac1 = pl.pallas_call(
        _encode_body,
        out_shape=[sd((N, H), _f32), sd((NA, HG), _f32), sd((NA, AC), _f32)],
    )(x, W1, r2(b1), r2(g1), r2(be1), W2, r2(b2), r2(g2), r2(be2),
      Wih, r2(bih), r2(bhh), Whh)

    zero = jnp.zeros((RPT, H), _f32)
    dma = pltpu.SemaphoreType.DMA

    conv2 = pl.kernel(
        _conv2_body,
        out_type=sd((2, NA, H), _f32),
        mesh=_sc_mesh(),
        scratch_types=[
            pltpu.VMEM((CT, B), jnp.int32),
            pltpu.VMEM((CT, B), jnp.int32),
            pltpu.VMEM((B, HG), _f32), pltpu.VMEM((B, HG), _f32),
            pltpu.VMEM((B, AC), _f32), pltpu.VMEM((B, AC), _f32),
            pltpu.VMEM((B, H), _f32), pltpu.VMEM((B, H), _f32),
            pltpu.VMEM_SHARED((NA, H), _f32),
            dma, dma, dma, dma, dma, dma,
        ],
        **_SC_PARAMS,
    )
    parts1 = conv2(_pad_idx(path_2[:, 0], NA - 1), _pad_idx(path_2[:, 1], NA - 1),
                   zx1, ac1, zero)

    h1, zx2, ac2 = pl.pallas_call(
        _finish_tables_body,
        out_shape=[sd((N, H), _f32), sd((NA, HG), _f32), sd((NA, AC), _f32)],
    )(parts1, h0, r2(gc1), r2(bc1), Wih, r2(bih), r2(bhh), Whh)

    conv3a = pl.kernel(
        _conv3a_body,
        out_type=[sd((PP, H), _f32), sd((PP, H), _f32)],
        mesh=_sc_mesh(),
        scratch_types=[
            pltpu.VMEM((CT, B), jnp.int32),
            pltpu.VMEM((CT, B), jnp.int32),
            pltpu.VMEM((B, HG), _f32), pltpu.VMEM((B, HG), _f32),
            pltpu.VMEM((B, AC), _f32), pltpu.VMEM((B, AC), _f32),
            pltpu.VMEM((B, H), _f32), pltpu.VMEM((B, H), _f32),
            pltpu.VMEM((B, H), _f32), pltpu.VMEM((B, H), _f32),
            dma, dma, dma, dma, dma, dma,
        ],
        **_SC_PARAMS,
    )
    h2p, c2p = conv3a(_pad_idx(path_3[:, 0], NA - 1),
                      _pad_idx(path_3[:, 1], NA - 1), zx2, ac2)

    RB = 2048
    g2m = pl.pallas_call(
        _gates_body,
        grid=(PP // RB,),
        in_specs=[
            pl.BlockSpec((RB, H), lambda i: (i, 0)),
            pl.BlockSpec((HG, H), lambda i: (0, 0)),
            pl.BlockSpec((1, HG), lambda i: (0, 0)),
        ],
        out_specs=pl.BlockSpec((RB, HG), lambda i: (i, 0)),
        out_shape=sd((PP, HG), _f32),
    )(h2p, Whh, r2(bhh))

    conv3b = pl.kernel(
        _conv3b_body,
        out_type=sd((2, NA, H), _f32),
        mesh=_sc_mesh(),
        scratch_types=[
            pltpu.VMEM((CT, B), jnp.int32),
            pltpu.VMEM((B, HG), _f32), pltpu.VMEM((B, HG), _f32),
            pltpu.VMEM((B, HG), _f32), pltpu.VMEM((B, HG), _f32),
            pltpu.VMEM((B, H), _f32), pltpu.VMEM((B, H), _f32),
            pltpu.VMEM((B, H), _f32), pltpu.VMEM((B, H), _f32),
            pltpu.VMEM_SHARED((NA, H), _f32),
            dma, dma, dma, dma, dma, dma, dma, dma,
        ],
        **_SC_PARAMS,
    )
    parts2 = conv3b(_pad_idx(path_3[:, 2], NA - 1), zx2, g2m, c2p, zero)

    out = pl.pallas_call(
        _head_body,
        out_shape=sd((G, C), _f32),
    )(parts2, h1, r2(gc2), r2(bc2), r2(batch), Wl1, r2(bl1), Wl2, r2(bl2))
    return out
